# baseline (speedup 1.0000x reference)
import functools
import math

import jax
import jax.numpy as jnp
from jax import lax
from jax.experimental import pallas as pl
from jax.experimental.pallas import tpu as pltpu
from jax.experimental.pallas import tpu_sc as plsc

PI = math.pi
SHIFT = math.log(2.0)
CUTOFF = 10.0

NC, NS = 2, 16
NW = NC * NS
CH = 128

FEAT = 128


def _proj_body(v_ref, w_ref, out_ref):
    out_ref[...] = jnp.dot(v_ref[...], w_ref[0],
                           preferred_element_type=jnp.float32)


def _softplus(x):
    return jnp.maximum(x, 0.0) + jnp.log(1.0 + jnp.exp(-jnp.abs(x)))


def _edge_body(d_ref, emb_ref, g_ref, w1_ref, b1_ref, w2_ref, b2_ref,
               sel_ref, msk_ref, one_ref, o_ref):
    x = jnp.dot(emb_ref[...], w1_ref[...],
                preferred_element_type=jnp.float32) + b1_ref[...]
    h = _softplus(x) - SHIFT
    w = jnp.dot(h, w2_ref[...],
                preferred_element_type=jnp.float32) + b2_ref[...]
    c2 = 0.5 * (jnp.cos(d_ref[0] * (PI / CUTOFF)) + 1.0)
    b = jnp.dot(sel_ref[...], c2, preferred_element_type=jnp.float32)
    c3 = jnp.dot(b * msk_ref[...], one_ref[...],
                 preferred_element_type=jnp.float32)
    o_ref[...] = g_ref[...] * (w * c3)


def _hull_body(fea_ref, g_ref, w1_ref, b1_ref, w2_ref, b2_ref, o_ref):
    x = jnp.dot(fea_ref[...], w1_ref[...],
                preferred_element_type=jnp.float32) + b1_ref[...]
    h = _softplus(x) - SHIFT
    w = jnp.dot(h, w2_ref[...],
                preferred_element_type=jnp.float32) + b2_ref[...]
    o_ref[...] = g_ref[...] * w


NBUF = 1


def _make_gather(cpw, npad):
    mesh = plsc.VectorSubcoreMesh(core_axis_name="c", subcore_axis_name="s")

    @functools.partial(
        pl.kernel,
        mesh=mesh,
        out_type=jax.ShapeDtypeStruct((npad, FEAT), jnp.float32),
        scratch_types=[
            pltpu.VMEM((cpw * CH,), jnp.int32),
            pltpu.VMEM((CH, FEAT), jnp.float32),
            pltpu.SemaphoreType.DMA,
        ],
    )
    def gather_k(table_hbm, jj_hbm, out_hbm, idx_v, rows_v, sem):
        wid = lax.axis_index("s") * NC + lax.axis_index("c")
        base = wid * cpw * CH
        pltpu.sync_copy(jj_hbm.at[pl.ds(base, cpw * CH)], idx_v)

        def body(k, carry):
            idx = idx_v.at[pl.ds(k * CH, CH)]
            pltpu.async_copy(table_hbm.at[idx], rows_v, sem).wait()
            pltpu.sync_copy(rows_v, out_hbm.at[pl.ds(base + k * CH, CH)])
            return carry

        lax.fori_loop(0, cpw, body, 0)

    return gather_k


def kernel(v, dist, dist_emb, edge_index, fea_hull, edge_index_hull,
           lin_w, mlp_w1, mlp_b1, mlp_w2, mlp_b2,
           lin_hull_w, mlp_hull_w1, mlp_hull_b1, mlp_hull_w2, mlp_hull_b2):
    n_nodes, hidden = v.shape
    n_edges = dist.shape[0]
    n_hull = fea_hull.shape[0]
    ngauss = dist_emb.shape[1]
    hull_dim = fea_hull.shape[1]

    wt_stack = jnp.stack([lin_w.T, lin_hull_w.T])
    table = pl.pallas_call(
        _proj_body,
        grid=(2,),
        in_specs=[
            pl.BlockSpec((n_nodes, hidden), lambda i: (0, 0)),
            pl.BlockSpec((1, hidden, FEAT), lambda i: (i, 0, 0)),
        ],
        out_specs=pl.BlockSpec((n_nodes, FEAT), lambda i: (i, 0)),
        out_shape=jax.ShapeDtypeStruct((2 * n_nodes, FEAT), jnp.float32),
    )(v, wt_stack)

    def run_gather(idx):
        n = idx.shape[0]
        cpw = -(-n // (NW * CH))
        npad = NW * cpw * CH
        jj = jnp.concatenate([idx, jnp.zeros((npad - n,), jnp.int32)])
        return _make_gather(cpw, npad)(table, jj)

    ge = run_gather(edge_index[0].astype(jnp.int32))
    gh = run_gather(edge_index_hull[0].astype(jnp.int32) + n_nodes)

    eb = 3200
    nq = eb // 128
    d2 = dist.reshape(n_edges // eb, nq, 128)
    rows = jax.lax.broadcasted_iota(jnp.int32, (eb, 128), 0)
    lanes = jax.lax.broadcasted_iota(jnp.int32, (eb, 128), 1)
    qsel = jax.lax.broadcasted_iota(jnp.int32, (eb, nq), 1)
    sel = (rows[:, :nq] // 128 == qsel).astype(jnp.float32)
    msk = (lanes == rows % 128).astype(jnp.float32)
    onecol = jnp.ones((128, 1), jnp.float32)
    e = pl.pallas_call(
        _edge_body,
        grid=(n_edges // eb,),
        in_specs=[
            pl.BlockSpec((1, nq, 128), lambda i: (i, 0, 0)),
            pl.BlockSpec((eb, ngauss), lambda i: (i, 0)),
            pl.BlockSpec((eb, FEAT), lambda i: (i, 0)),
            pl.BlockSpec((ngauss, FEAT), lambda i: (0, 0)),
            pl.BlockSpec((1, FEAT), lambda i: (0, 0)),
            pl.BlockSpec((FEAT, FEAT), lambda i: (0, 0)),
            pl.BlockSpec((1, FEAT), lambda i: (0, 0)),
            pl.BlockSpec((eb, nq), lambda i: (0, 0)),
            pl.BlockSpec((eb, 128), lambda i: (0, 0)),
            pl.BlockSpec((128, 1), lambda i: (0, 0)),
        ],
        out_specs=pl.BlockSpec((eb, FEAT), lambda i: (i, 0)),
        out_shape=jax.ShapeDtypeStruct((n_edges, FEAT), jnp.float32),
    )(d2, dist_emb, ge, mlp_w1.T, mlp_b1.reshape(1, FEAT),
      mlp_w2.T, mlp_b2.reshape(1, FEAT), sel, msk, onecol)

    hb = 2000
    e_hull = pl.pallas_call(
        _hull_body,
        grid=(n_hull // hb,),
        in_specs=[
            pl.BlockSpec((hb, hull_dim), lambda i: (i, 0)),
            pl.BlockSpec((hb, FEAT), lambda i: (i, 0)),
            pl.BlockSpec((hull_dim, FEAT), lambda i: (0, 0)),
            pl.BlockSpec((1, FEAT), lambda i: (0, 0)),
            pl.BlockSpec((FEAT, FEAT), lambda i: (0, 0)),
            pl.BlockSpec((1, FEAT), lambda i: (0, 0)),
        ],
        out_specs=pl.BlockSpec((hb, FEAT), lambda i: (i, 0)),
        out_shape=jax.ShapeDtypeStruct((n_hull, FEAT), jnp.float32),
    )(fea_hull, gh, mlp_hull_w1.T, mlp_hull_b1.reshape(1, FEAT),
      mlp_hull_w2.T, mlp_hull_b2.reshape(1, FEAT))

    return (e, e_hull)


# speedup vs baseline: 1.2341x; 1.2341x over previous
import functools
import math

import jax
import jax.numpy as jnp
from jax import lax
from jax.experimental import pallas as pl
from jax.experimental.pallas import tpu as pltpu
from jax.experimental.pallas import tpu_sc as plsc

PI = math.pi
SHIFT = math.log(2.0)
CUTOFF = 10.0

NC, NS = 2, 16
NW = NC * NS
CH = 128

FEAT = 128


def _proj_body(v_ref, w_ref, out_ref):
    out_ref[...] = jnp.dot(v_ref[...], w_ref[0],
                           preferred_element_type=jnp.float32)


def _softplus(x):
    return jnp.maximum(x, 0.0) + jnp.log(1.0 + jnp.exp(-jnp.abs(x)))


def _edge_body(d_ref, emb_ref, g_ref, w1_ref, b1_ref, w2_ref, b2_ref,
               sel_ref, msk_ref, one_ref, o_ref):
    x = jnp.dot(emb_ref[...], w1_ref[...],
                preferred_element_type=jnp.float32) + b1_ref[...]
    h = _softplus(x) - SHIFT
    w = jnp.dot(h, w2_ref[...],
                preferred_element_type=jnp.float32) + b2_ref[...]
    c2 = 0.5 * (jnp.cos(d_ref[0] * (PI / CUTOFF)) + 1.0)
    b = jnp.dot(sel_ref[...], c2, preferred_element_type=jnp.float32)
    c3 = jnp.dot(b * msk_ref[...], one_ref[...],
                 preferred_element_type=jnp.float32)
    o_ref[...] = g_ref[...] * (w * c3)


def _hull_body(fea_ref, g_ref, w1_ref, b1_ref, w2_ref, b2_ref, o_ref):
    x = jnp.dot(fea_ref[...], w1_ref[...],
                preferred_element_type=jnp.float32) + b1_ref[...]
    h = _softplus(x) - SHIFT
    w = jnp.dot(h, w2_ref[...],
                preferred_element_type=jnp.float32) + b2_ref[...]
    o_ref[...] = g_ref[...] * w


def _make_gather(ca, cb, alloc):
    mesh = plsc.VectorSubcoreMesh(core_axis_name="c", subcore_axis_name="s")
    mn, mx = min(ca, cb), max(ca, cb)

    @functools.partial(
        pl.kernel,
        mesh=mesh,
        out_type=jax.ShapeDtypeStruct((alloc, FEAT), jnp.float32),
        scratch_types=[
            pltpu.VMEM((mx * CH,), jnp.int32),
            pltpu.VMEM((CH, FEAT), jnp.float32),
            pltpu.SemaphoreType.DMA,
        ],
    )
    def gather_k(table_hbm, jj_hbm, out_hbm, idx_v, rows_v, sem):
        c = lax.axis_index("c")
        s = lax.axis_index("s")
        my = jnp.where(c == 0, ca, cb)
        base = jnp.where(c == 0, s * (ca * CH),
                         NS * (ca * CH) + s * (cb * CH))
        pltpu.sync_copy(jj_hbm.at[pl.ds(base, mx * CH)], idx_v)

        def body(k, carry):
            idx = idx_v.at[pl.ds(k * CH, CH)]
            pltpu.async_copy(table_hbm.at[idx], rows_v, sem).wait()
            pltpu.sync_copy(rows_v, out_hbm.at[pl.ds(base + k * CH, CH)])
            return carry

        lax.fori_loop(0, mn, body, 0)

        @pl.when(my > mn)
        def _():
            lax.fori_loop(mn, mx, body, 0)

    return gather_k


def kernel(v, dist, dist_emb, edge_index, fea_hull, edge_index_hull,
           lin_w, mlp_w1, mlp_b1, mlp_w2, mlp_b2,
           lin_hull_w, mlp_hull_w1, mlp_hull_b1, mlp_hull_w2, mlp_hull_b2):
    n_nodes, hidden = v.shape
    n_edges = dist.shape[0]
    n_hull = fea_hull.shape[0]
    ngauss = dist_emb.shape[1]
    hull_dim = fea_hull.shape[1]

    wt_stack = jnp.stack([lin_w.T, lin_hull_w.T])
    table = pl.pallas_call(
        _proj_body,
        grid=(2,),
        in_specs=[
            pl.BlockSpec((n_nodes, hidden), lambda i: (0, 0)),
            pl.BlockSpec((1, hidden, FEAT), lambda i: (i, 0, 0)),
        ],
        out_specs=pl.BlockSpec((n_nodes, FEAT), lambda i: (i, 0)),
        out_shape=jax.ShapeDtypeStruct((2 * n_nodes, FEAT), jnp.float32),
    )(v, wt_stack)

    j = edge_index[0].astype(jnp.int32)
    j_ = edge_index_hull[0].astype(jnp.int32) + n_nodes
    ntot = n_edges + n_hull
    cpw = -(-ntot // (NW * CH))
    ca, cb = cpw - 16, cpw + 16
    alloc = (NS * (ca + cb) + (max(ca, cb) - min(ca, cb))) * CH
    jj = jnp.concatenate(
        [j, j_, jnp.zeros((alloc - ntot,), jnp.int32)])
    g = _make_gather(ca, cb, alloc)(table, jj)
    ge = g
    gh = g

    eb = 3200
    nq = eb // 128
    d2 = dist.reshape(n_edges // eb, nq, 128)
    rows = jax.lax.broadcasted_iota(jnp.int32, (eb, 128), 0)
    lanes = jax.lax.broadcasted_iota(jnp.int32, (eb, 128), 1)
    qsel = jax.lax.broadcasted_iota(jnp.int32, (eb, nq), 1)
    sel = (rows[:, :nq] // 128 == qsel).astype(jnp.float32)
    msk = (lanes == rows % 128).astype(jnp.float32)
    onecol = jnp.ones((128, 1), jnp.float32)
    e = pl.pallas_call(
        _edge_body,
        grid=(n_edges // eb,),
        in_specs=[
            pl.BlockSpec((1, nq, 128), lambda i: (i, 0, 0)),
            pl.BlockSpec((eb, ngauss), lambda i: (i, 0)),
            pl.BlockSpec((eb, FEAT), lambda i: (i, 0)),
            pl.BlockSpec((ngauss, FEAT), lambda i: (0, 0)),
            pl.BlockSpec((1, FEAT), lambda i: (0, 0)),
            pl.BlockSpec((FEAT, FEAT), lambda i: (0, 0)),
            pl.BlockSpec((1, FEAT), lambda i: (0, 0)),
            pl.BlockSpec((eb, nq), lambda i: (0, 0)),
            pl.BlockSpec((eb, 128), lambda i: (0, 0)),
            pl.BlockSpec((128, 1), lambda i: (0, 0)),
        ],
        out_specs=pl.BlockSpec((eb, FEAT), lambda i: (i, 0)),
        out_shape=jax.ShapeDtypeStruct((n_edges, FEAT), jnp.float32),
    )(d2, dist_emb, ge, mlp_w1.T, mlp_b1.reshape(1, FEAT),
      mlp_w2.T, mlp_b2.reshape(1, FEAT), sel, msk, onecol)

    hb = 2000
    hull_off = n_edges // hb
    e_hull = pl.pallas_call(
        _hull_body,
        grid=(n_hull // hb,),
        in_specs=[
            pl.BlockSpec((hb, hull_dim), lambda i: (i, 0)),
            pl.BlockSpec((hb, FEAT), lambda i: (i + hull_off, 0)),
            pl.BlockSpec((hull_dim, FEAT), lambda i: (0, 0)),
            pl.BlockSpec((1, FEAT), lambda i: (0, 0)),
            pl.BlockSpec((FEAT, FEAT), lambda i: (0, 0)),
            pl.BlockSpec((1, FEAT), lambda i: (0, 0)),
        ],
        out_specs=pl.BlockSpec((hb, FEAT), lambda i: (i, 0)),
        out_shape=jax.ShapeDtypeStruct((n_hull, FEAT), jnp.float32),
    )(fea_hull, gh, mlp_hull_w1.T, mlp_hull_b1.reshape(1, FEAT),
      mlp_hull_w2.T, mlp_hull_b2.reshape(1, FEAT))

    return (e, e_hull)


# speedup vs baseline: 1.2772x; 1.0349x over previous
import functools
import math

import jax
import jax.numpy as jnp
from jax import lax
from jax.experimental import pallas as pl
from jax.experimental.pallas import tpu as pltpu
from jax.experimental.pallas import tpu_sc as plsc

PI = math.pi
SHIFT = math.log(2.0)
CUTOFF = 10.0

NC, NS = 2, 16
NW = NC * NS
CH = 128

FEAT = 128


def _proj_body(v_ref, w_ref, out_ref):
    out_ref[...] = jnp.dot(v_ref[...], w_ref[0],
                           preferred_element_type=jnp.float32)


def _softplus(x):
    return jnp.maximum(x, 0.0) + jnp.log(1.0 + jnp.exp(-jnp.abs(x)))


def _edge_body(d_ref, emb_ref, g_ref, w1_ref, b1_ref, w2_ref, b2_ref,
               sel_ref, msk_ref, one_ref, o_ref):
    x = jnp.dot(emb_ref[...], w1_ref[...],
                preferred_element_type=jnp.float32) + b1_ref[...]
    h = _softplus(x) - SHIFT
    w = jnp.dot(h, w2_ref[...],
                preferred_element_type=jnp.float32) + b2_ref[...]
    c2 = 0.5 * (jnp.cos(d_ref[0] * (PI / CUTOFF)) + 1.0)
    b = jnp.dot(sel_ref[...], c2, preferred_element_type=jnp.float32)
    c3 = jnp.dot(b * msk_ref[...], one_ref[...],
                 preferred_element_type=jnp.float32)
    o_ref[...] = g_ref[...] * (w * c3)


def _hull_body(fea_ref, g_ref, w1_ref, b1_ref, w2_ref, b2_ref, o_ref):
    x = jnp.dot(fea_ref[...], w1_ref[...],
                preferred_element_type=jnp.float32) + b1_ref[...]
    h = _softplus(x) - SHIFT
    w = jnp.dot(h, w2_ref[...],
                preferred_element_type=jnp.float32) + b2_ref[...]
    o_ref[...] = g_ref[...] * w


def _make_gather(ca, cb, alloc):
    mesh = plsc.VectorSubcoreMesh(core_axis_name="c", subcore_axis_name="s")
    mn, mx = min(ca, cb), max(ca, cb)

    @functools.partial(
        pl.kernel,
        mesh=mesh,
        out_type=jax.ShapeDtypeStruct((alloc, FEAT), jnp.float32),
        scratch_types=[
            pltpu.VMEM((mx * CH,), jnp.int32),
            pltpu.VMEM((CH, FEAT), jnp.float32),
            pltpu.SemaphoreType.DMA,
        ],
    )
    def gather_k(table_hbm, jj_hbm, out_hbm, idx_v, rows_v, sem):
        c = lax.axis_index("c")
        s = lax.axis_index("s")
        my = jnp.where(c == 0, ca, cb)
        base = jnp.where(c == 0, s * (ca * CH),
                         NS * (ca * CH) + s * (cb * CH))
        pltpu.sync_copy(jj_hbm.at[pl.ds(base, mx * CH)], idx_v)

        def body(k, carry):
            idx = idx_v.at[pl.ds(k * CH, CH)]
            pltpu.async_copy(table_hbm.at[idx], rows_v, sem).wait()
            pltpu.sync_copy(rows_v, out_hbm.at[pl.ds(base + k * CH, CH)])
            return carry

        lax.fori_loop(0, mn, body, 0)

        @pl.when(my > mn)
        def _():
            lax.fori_loop(mn, mx, body, 0)

    return gather_k


def kernel(v, dist, dist_emb, edge_index, fea_hull, edge_index_hull,
           lin_w, mlp_w1, mlp_b1, mlp_w2, mlp_b2,
           lin_hull_w, mlp_hull_w1, mlp_hull_b1, mlp_hull_w2, mlp_hull_b2):
    n_nodes, hidden = v.shape
    n_edges = dist.shape[0]
    n_hull = fea_hull.shape[0]
    ngauss = dist_emb.shape[1]
    hull_dim = fea_hull.shape[1]

    wt_stack = jnp.stack([lin_w.T, lin_hull_w.T])
    table = pl.pallas_call(
        _proj_body,
        grid=(2,),
        in_specs=[
            pl.BlockSpec((n_nodes, hidden), lambda i: (0, 0)),
            pl.BlockSpec((1, hidden, FEAT), lambda i: (i, 0, 0)),
        ],
        out_specs=pl.BlockSpec((n_nodes, FEAT), lambda i: (i, 0)),
        out_shape=jax.ShapeDtypeStruct((2 * n_nodes, FEAT), jnp.float32),
    )(v, wt_stack)

    j = edge_index[0].astype(jnp.int32)
    j_ = edge_index_hull[0].astype(jnp.int32) + n_nodes
    ntot = n_edges + n_hull
    cpw = -(-ntot // (NW * CH))
    ca, cb = cpw + 16, cpw - 16
    alloc = (NS * (ca + cb) + (max(ca, cb) - min(ca, cb))) * CH
    jj = jnp.concatenate(
        [j, j_, jnp.zeros((alloc - ntot,), jnp.int32)])
    g = _make_gather(ca, cb, alloc)(table, jj)
    ge = g
    gh = g

    eb = 3200
    nq = eb // 128
    d2 = dist.reshape(n_edges // eb, nq, 128)
    rows = jax.lax.broadcasted_iota(jnp.int32, (eb, 128), 0)
    lanes = jax.lax.broadcasted_iota(jnp.int32, (eb, 128), 1)
    qsel = jax.lax.broadcasted_iota(jnp.int32, (eb, nq), 1)
    sel = (rows[:, :nq] // 128 == qsel).astype(jnp.float32)
    msk = (lanes == rows % 128).astype(jnp.float32)
    onecol = jnp.ones((128, 1), jnp.float32)
    e = pl.pallas_call(
        _edge_body,
        grid=(n_edges // eb,),
        in_specs=[
            pl.BlockSpec((1, nq, 128), lambda i: (i, 0, 0)),
            pl.BlockSpec((eb, ngauss), lambda i: (i, 0)),
            pl.BlockSpec((eb, FEAT), lambda i: (i, 0)),
            pl.BlockSpec((ngauss, FEAT), lambda i: (0, 0)),
            pl.BlockSpec((1, FEAT), lambda i: (0, 0)),
            pl.BlockSpec((FEAT, FEAT), lambda i: (0, 0)),
            pl.BlockSpec((1, FEAT), lambda i: (0, 0)),
            pl.BlockSpec((eb, nq), lambda i: (0, 0)),
            pl.BlockSpec((eb, 128), lambda i: (0, 0)),
            pl.BlockSpec((128, 1), lambda i: (0, 0)),
        ],
        out_specs=pl.BlockSpec((eb, FEAT), lambda i: (i, 0)),
        out_shape=jax.ShapeDtypeStruct((n_edges, FEAT), jnp.float32),
    )(d2, dist_emb, ge, mlp_w1.T, mlp_b1.reshape(1, FEAT),
      mlp_w2.T, mlp_b2.reshape(1, FEAT), sel, msk, onecol)

    hb = 2000
    hull_off = n_edges // hb
    e_hull = pl.pallas_call(
        _hull_body,
        grid=(n_hull // hb,),
        in_specs=[
            pl.BlockSpec((hb, hull_dim), lambda i: (i, 0)),
            pl.BlockSpec((hb, FEAT), lambda i: (i + hull_off, 0)),
            pl.BlockSpec((hull_dim, FEAT), lambda i: (0, 0)),
            pl.BlockSpec((1, FEAT), lambda i: (0, 0)),
            pl.BlockSpec((FEAT, FEAT), lambda i: (0, 0)),
            pl.BlockSpec((1, FEAT), lambda i: (0, 0)),
        ],
        out_specs=pl.BlockSpec((hb, FEAT), lambda i: (i, 0)),
        out_shape=jax.ShapeDtypeStruct((n_hull, FEAT), jnp.float32),
    )(fea_hull, gh, mlp_hull_w1.T, mlp_hull_b1.reshape(1, FEAT),
      mlp_hull_w2.T, mlp_hull_b2.reshape(1, FEAT))

    return (e, e_hull)
